# Initial kernel scaffold; baseline (speedup 1.0000x reference)
import jax
import jax.numpy as jnp
from jax.experimental import pallas as pl


def kernel(node_feats, edge_feats, edge_index, W1_0, W2_0, W3_0, W1_1, W2_1, W3_1):
    raise NotImplementedError("write your pallas kernel here")


# speedup vs baseline: 7.6519x; 7.6519x over previous
import functools

import jax
import jax.numpy as jnp
from jax import lax
from jax.experimental import pallas as pl
from jax.experimental.pallas import tpu as pltpu
from jax.experimental.pallas import tpu_sc as plsc

N = 10000
E = 320000
D = 128
SLOPE = (1.0 / 8.0 + 1.0 / 3.0) / 2.0

NC = 2
NS = 16
NW = NC * NS
EPW = E // NW
CHUNK = 80
NCHUNK = EPW // CHUNK
NPAIR = (NCHUNK - 1) // 2
NPAD = 10240
RPT = NPAD // NS
DEG_PAD = 10240
DEG_RPT = DEG_PAD // NS

_MESH = plsc.VectorSubcoreMesh(core_axis_name="c", subcore_axis_name="s")


def _sc_ef_deg_body(ef_hbm, dst_hbm, z2_hbm, z1_hbm, efagg_out, deg_out,
                    dstv, rows, ones1, acc_ef, acc_deg, *sems):
    c = lax.axis_index("c")
    s = lax.axis_index("s")
    wid = c * NS + s

    one = jnp.ones((16,), jnp.float32)
    for i in range(CHUNK // 16):
        ones1[pl.ds(i * 16, 16)] = one

    pltpu.sync_copy(z2_hbm.at[pl.ds(s * RPT, RPT)],
                    acc_ef.at[pl.ds(s * RPT, RPT)])
    pltpu.sync_copy(z1_hbm.at[pl.ds(s * DEG_RPT, DEG_RPT)],
                    acc_deg.at[pl.ds(s * DEG_RPT, DEG_RPT)])
    plsc.subcore_barrier()

    pltpu.sync_copy(dst_hbm.at[wid], dstv)

    buf0 = rows.at[0]
    buf1 = rows.at[1]
    gsem0, gsem1, ssem0, ssem1, osem = sems

    def fire_g(j, buf, gs):
        pltpu.async_copy(ef_hbm.at[pl.ds(wid * EPW + j * CHUNK, CHUNK)],
                         buf, gs)

    def drain_g(buf, gs):
        pltpu.make_async_copy(ef_hbm.at[pl.ds(0, CHUNK)], buf, gs).wait()

    def fire_s(j, buf, ss):
        pltpu.async_copy(buf, acc_ef.at[dstv.at[j]], ss, add=True)
        pltpu.async_copy(ones1, acc_deg.at[dstv.at[j]], osem, add=True)

    def drain_s(buf, ss):
        pltpu.make_async_copy(buf, acc_ef.at[dstv.at[0]], ss).wait()

    def drain_o():
        pltpu.make_async_copy(ones1, acc_deg.at[dstv.at[0]], osem).wait()

    fire_g(0, buf0, gsem0)
    fire_g(1, buf1, gsem1)

    def pair(t, carry):
        j0 = 2 * t
        drain_g(buf0, gsem0)
        fire_s(j0, buf0, ssem0)
        drain_g(buf1, gsem1)
        fire_s(j0 + 1, buf1, ssem1)
        drain_s(buf0, ssem0)
        fire_g(j0 + 2, buf0, gsem0)
        drain_s(buf1, ssem1)
        drain_o()
        drain_o()

        @pl.when(j0 + 3 < NCHUNK)
        def _():
            fire_g(j0 + 3, buf1, gsem1)

        return carry

    lax.fori_loop(0, NPAIR, pair, 0)
    drain_g(buf0, gsem0)
    fire_s(NCHUNK - 1, buf0, ssem0)
    drain_s(buf0, ssem0)
    drain_o()
    plsc.subcore_barrier()

    pltpu.sync_copy(acc_ef.at[pl.ds(s * RPT, RPT)],
                    efagg_out.at[c, pl.ds(s * RPT, RPT)])
    pltpu.sync_copy(acc_deg.at[pl.ds(s * DEG_RPT, DEG_RPT)],
                    deg_out.at[c, pl.ds(s * DEG_RPT, DEG_RPT)])


_sc_ef_deg = pl.kernel(
    _sc_ef_deg_body,
    out_type=[jax.ShapeDtypeStruct((NC, NPAD, D), jnp.float32),
              jax.ShapeDtypeStruct((NC, DEG_PAD), jnp.float32)],
    mesh=_MESH,
    scratch_types=[
        pltpu.VMEM((NCHUNK, CHUNK), jnp.int32),
        pltpu.VMEM((2, CHUNK, D), jnp.float32),
        pltpu.VMEM((CHUNK,), jnp.float32),
        pltpu.VMEM_SHARED((NPAD, D), jnp.float32),
        pltpu.VMEM_SHARED((DEG_PAD,), jnp.float32),
        pltpu.SemaphoreType.DMA,
        pltpu.SemaphoreType.DMA,
        pltpu.SemaphoreType.DMA,
        pltpu.SemaphoreType.DMA,
        pltpu.SemaphoreType.DMA,
    ],
)


def _sc_spmm_body(x_hbm, src_hbm, dst_hbm, z2_hbm, xagg_out,
                  srcv, dstv, rows, acc, *sems):
    c = lax.axis_index("c")
    s = lax.axis_index("s")
    wid = c * NS + s

    pltpu.sync_copy(z2_hbm.at[pl.ds(s * RPT, RPT)],
                    acc.at[pl.ds(s * RPT, RPT)])
    plsc.subcore_barrier()

    pltpu.sync_copy(src_hbm.at[pl.ds(wid * EPW, EPW)], srcv)
    pltpu.sync_copy(dst_hbm.at[wid], dstv)

    buf0 = rows.at[0]
    buf1 = rows.at[1]
    gsem0, gsem1, ssem0, ssem1 = sems

    def fire_g(j, buf, gs):
        pltpu.async_copy(x_hbm.at[srcv.at[pl.ds(j * CHUNK, CHUNK)]], buf, gs)

    def drain_g(buf, gs):
        pltpu.make_async_copy(x_hbm.at[pl.ds(0, CHUNK)], buf, gs).wait()

    def fire_s(j, buf, ss):
        pltpu.async_copy(buf, acc.at[dstv.at[j]], ss, add=True)

    def drain_s(buf, ss):
        pltpu.make_async_copy(buf, acc.at[dstv.at[0]], ss).wait()

    fire_g(0, buf0, gsem0)
    fire_g(1, buf1, gsem1)

    def pair(t, carry):
        j0 = 2 * t
        drain_g(buf0, gsem0)
        fire_s(j0, buf0, ssem0)
        drain_g(buf1, gsem1)
        fire_s(j0 + 1, buf1, ssem1)
        drain_s(buf0, ssem0)
        fire_g(j0 + 2, buf0, gsem0)
        drain_s(buf1, ssem1)

        @pl.when(j0 + 3 < NCHUNK)
        def _():
            fire_g(j0 + 3, buf1, gsem1)

        return carry

    lax.fori_loop(0, NPAIR, pair, 0)
    drain_g(buf0, gsem0)
    fire_s(NCHUNK - 1, buf0, ssem0)
    drain_s(buf0, ssem0)
    plsc.subcore_barrier()

    pltpu.sync_copy(acc.at[pl.ds(s * RPT, RPT)],
                    xagg_out.at[c, pl.ds(s * RPT, RPT)])


_sc_spmm = pl.kernel(
    _sc_spmm_body,
    out_type=jax.ShapeDtypeStruct((NC, NPAD, D), jnp.float32),
    mesh=_MESH,
    scratch_types=[
        pltpu.VMEM((EPW,), jnp.int32),
        pltpu.VMEM((NCHUNK, CHUNK), jnp.int32),
        pltpu.VMEM((2, CHUNK, D), jnp.float32),
        pltpu.VMEM_SHARED((NPAD, D), jnp.float32),
        pltpu.SemaphoreType.DMA,
        pltpu.SemaphoreType.DMA,
        pltpu.SemaphoreType.DMA,
        pltpu.SemaphoreType.DMA,
    ],
)


BLK = 1000


def _dense_body(x_ref, xagg_ref, efagg_ref, deg_ref, w1_ref, w2_ref, w3_ref,
                out_ref):
    x = x_ref[...]
    agg = xagg_ref[0] + xagg_ref[1] + efagg_ref[0] + efagg_ref[1]
    deg = deg_ref[...]
    degs = deg[:, 0:1] + deg[:, 1:2]
    dn = (((1,), (1,)), ((), ()))
    neigh = lax.dot_general(agg, w1_ref[...], dn,
                            preferred_element_type=jnp.float32)
    neigh = neigh / jnp.maximum(degs, 1.0)
    s2 = lax.dot_general(x, w2_ref[...], dn,
                         preferred_element_type=jnp.float32)
    s3 = lax.dot_general(x, w3_ref[...], dn,
                         preferred_element_type=jnp.float32)
    h = neigh + jnp.where(degs == 0.0, s3, s2)
    out_ref[...] = jnp.where(h >= 0.0, h, SLOPE * h)


def _dense(x, xagg, efagg, deg2, w1, w2, w3):
    return pl.pallas_call(
        _dense_body,
        grid=(N // BLK,),
        in_specs=[
            pl.BlockSpec((BLK, D), lambda i: (i, 0)),
            pl.BlockSpec((NC, BLK, D), lambda i: (0, i, 0)),
            pl.BlockSpec((NC, BLK, D), lambda i: (0, i, 0)),
            pl.BlockSpec((BLK, NC), lambda i: (i, 0)),
            pl.BlockSpec((D, D), lambda i: (0, 0)),
            pl.BlockSpec((D, D), lambda i: (0, 0)),
            pl.BlockSpec((D, D), lambda i: (0, 0)),
        ],
        out_specs=pl.BlockSpec((BLK, D), lambda i: (i, 0)),
        out_shape=jax.ShapeDtypeStruct((N, D), jnp.float32),
    )(x, xagg, efagg, deg2, w1, w2, w3)


def kernel(node_feats, edge_feats, edge_index, W1_0, W2_0, W3_0,
           W1_1, W2_1, W3_1):
    src_flat = edge_index[0]
    dst3 = edge_index[1].reshape(NW, NCHUNK, CHUNK)

    z2 = jnp.zeros((NPAD, D), jnp.float32)
    z1 = jnp.zeros((DEG_PAD,), jnp.float32)

    efagg_p, deg_p = _sc_ef_deg(edge_feats, dst3, z2, z1)
    deg2 = deg_p[:, :N].T

    xagg0 = _sc_spmm(node_feats, src_flat, dst3, z2)
    h0 = _dense(node_feats, xagg0, efagg_p, deg2, W1_0, W2_0, W3_0)
    xagg1 = _sc_spmm(h0, src_flat, dst3, z2)
    h1 = _dense(h0, xagg1, efagg_p, deg2, W1_1, W2_1, W3_1)
    return h1
